# Initial kernel scaffold; baseline (speedup 1.0000x reference)
import jax
import jax.numpy as jnp
from jax.experimental import pallas as pl


def kernel(x, edge_index, W1, a_src1, a_dst1, b1, W2, a_src2, a_dst2, b2):
    raise NotImplementedError("write your pallas kernel here")


# speedup vs baseline: 38.5700x; 38.5700x over previous
import functools

import jax
import jax.numpy as jnp
from jax.experimental import pallas as pl
from jax.experimental.pallas import tpu as pltpu
from jax.experimental.pallas import tpu_sc as plsc

_N = 10000
_E = 320000
_NFEAT = 128
_NCLASS = 40
_H1, _C1 = 8, 8

_NC, _NS = 2, 16
_NW = _NC * _NS
_CHUNK = 128
_ROWS_PER_TILE = 640
_NP = _NW * _ROWS_PER_TILE // 2
_ETOT = _E + _N
_CHUNKS_PER_TILE = -(-_ETOT // (_NW * _CHUNK))
_NCHUNK = _NW * _CHUNKS_PER_TILE
_EP = _NCHUNK * _CHUNK


def _tc_tables1(x_ref, w1_ref, ms_ref, md_ref, a_ref, b_ref):
    h = jnp.dot(x_ref[...], w1_ref[...], preferred_element_type=jnp.float32)
    asb = jnp.dot(h, ms_ref[...], preferred_element_type=jnp.float32)
    adb = jnp.dot(h, md_ref[...], preferred_element_type=jnp.float32)
    a_ref[...] = jnp.concatenate([h, asb], axis=1)
    b_ref[...] = jnp.concatenate(
        [adb, jnp.zeros((adb.shape[0], 64), jnp.float32)], axis=1)


def _tc_combine1(acc_ref, b1_ref, w2_ref, s2_ref, d2_ref, a_ref, b_ref):
    a = acc_ref[0] + acc_ref[1]
    num = a[:, :64]
    den = a[:, 64:]
    x1 = num / (den + 1e-16) + b1_ref[...]
    x1 = jnp.where(x1 > 0, x1, jnp.exp(jnp.minimum(x1, 0.0)) - 1.0)
    h2 = jnp.dot(x1, w2_ref[...], preferred_element_type=jnp.float32)
    as2 = jnp.dot(h2, s2_ref[...], preferred_element_type=jnp.float32)
    ad2 = jnp.dot(h2, d2_ref[...], preferred_element_type=jnp.float32)
    zeros8 = jnp.zeros((h2.shape[0], 8), jnp.float32)
    zeros64 = jnp.zeros((h2.shape[0], 64), jnp.float32)
    a_ref[...] = jnp.concatenate([h2, zeros8, as2, zeros64], axis=1)
    b_ref[...] = jnp.concatenate(
        [ad2, jnp.zeros((ad2.shape[0], 112), jnp.float32)], axis=1)


def _tc_combine2(acc_ref, b2_ref, o_ref):
    a = acc_ref[0] + acc_ref[1]
    num = a[:, :_NCLASS]
    den = a[:, 48:49]
    o_ref[...] = num / (den + 1e-16) + b2_ref[...]


def _edge_body1(ga, gb, e):
    for p in range(4):
        va = ga[e, pl.ds(64 + 16 * p, 16)]
        vd = gb[e, pl.ds(16 * p, 16)]
        s = va + vd
        w = jnp.exp(jnp.where(s > 0, s, 0.2 * s))
        ga[e, pl.ds(16 * p, 16)] = ga[e, pl.ds(16 * p, 16)] * w
        ga[e, pl.ds(64 + 16 * p, 16)] = w


def _edge_body2(ga, gb, e):
    s = ga[e, pl.ds(48, 16)] + gb[e, pl.ds(0, 16)]
    w = jnp.exp(jnp.where(s > 0, s, 0.2 * s))
    for p in range(3):
        ga[e, pl.ds(16 * p, 16)] = ga[e, pl.ds(16 * p, 16)] * w
    ga[e, pl.ds(48, 16)] = w


def _sc_edge_pass(a_hbm, b_hbm, src_hbm, dst_hbm, zero_hbm, out_hbm,
                  sidx, didx, ga, gb, acc, *, edge_body):
    cid = jax.lax.axis_index("c")
    sid = jax.lax.axis_index("s")
    wid = sid * _NC + cid

    pltpu.sync_copy(zero_hbm, acc.at[pl.ds(sid * _ROWS_PER_TILE, _ROWS_PER_TILE)])
    plsc.subcore_barrier()

    @pl.loop(0, _CHUNKS_PER_TILE)
    def _(j):
        c = wid * _CHUNKS_PER_TILE + j
        pltpu.sync_copy(src_hbm.at[c], sidx)
        pltpu.sync_copy(dst_hbm.at[c], didx)
        pltpu.sync_copy(a_hbm.at[sidx], ga)
        pltpu.sync_copy(b_hbm.at[didx], gb)

        @pl.loop(0, _CHUNK)
        def _(e):
            edge_body(ga, gb, e)

        pltpu.sync_copy(ga, acc.at[didx], add=True)

    plsc.subcore_barrier()
    pltpu.sync_copy(
        acc.at[pl.ds(sid * _ROWS_PER_TILE, _ROWS_PER_TILE)],
        out_hbm.at[pl.ds((cid * _NP + sid * _ROWS_PER_TILE), _ROWS_PER_TILE)],
    )


def _sc_layer(a_tab, b_tab, src2d, dst2d, edge_body):
    zero = jnp.zeros((_ROWS_PER_TILE, 128), jnp.float32)
    mesh = plsc.VectorSubcoreMesh(core_axis_name="c", subcore_axis_name="s")
    kern = pl.kernel(
        functools.partial(_sc_edge_pass, edge_body=edge_body),
        out_type=jax.ShapeDtypeStruct((2 * _NP, 128), jnp.float32),
        mesh=mesh,
        scratch_types=[
            pltpu.VMEM((_CHUNK,), jnp.int32),
            pltpu.VMEM((_CHUNK,), jnp.int32),
            pltpu.VMEM((_CHUNK, 128), jnp.float32),
            pltpu.VMEM((_CHUNK, 128), jnp.float32),
            pltpu.VMEM_SHARED((_NP, 128), jnp.float32),
        ],
    )
    out = kern(a_tab, b_tab, src2d, dst2d, zero)
    return out.reshape(2, _NP, 128)


def kernel(x, edge_index, W1, a_src1, a_dst1, b1, W2, a_src2, a_dst2, b2):
    n = x.shape[0]
    loops = jnp.arange(n, dtype=jnp.int32)
    padi = jnp.full((_EP - _ETOT,), n, dtype=jnp.int32)
    src = jnp.concatenate([edge_index[0], loops, padi]).reshape(_NCHUNK, _CHUNK)
    dst = jnp.concatenate([edge_index[1], loops, padi]).reshape(_NCHUNK, _CHUNK)

    x_pad = jnp.zeros((_NP, _NFEAT), jnp.float32).at[:n].set(x)

    lane64 = jnp.arange(64)
    headmask = (lane64[:, None] // _C1 == lane64[None, :] // _C1)
    ms = jnp.where(headmask, a_src1.reshape(64)[:, None], 0.0)
    md = jnp.where(headmask, a_dst1.reshape(64)[:, None], 0.0)
    s2 = jnp.broadcast_to(a_src2[0][:, None], (_NCLASS, 16)).astype(jnp.float32)
    d2 = jnp.broadcast_to(a_dst2[0][:, None], (_NCLASS, 16)).astype(jnp.float32)

    blk = 512
    grid = (_NP // blk,)

    a1, btab1 = pl.pallas_call(
        _tc_tables1,
        grid=grid,
        in_specs=[
            pl.BlockSpec((blk, _NFEAT), lambda i: (i, 0)),
            pl.BlockSpec((_NFEAT, 64), lambda i: (0, 0)),
            pl.BlockSpec((64, 64), lambda i: (0, 0)),
            pl.BlockSpec((64, 64), lambda i: (0, 0)),
        ],
        out_specs=[
            pl.BlockSpec((blk, 128), lambda i: (i, 0)),
            pl.BlockSpec((blk, 128), lambda i: (i, 0)),
        ],
        out_shape=[
            jax.ShapeDtypeStruct((_NP, 128), jnp.float32),
            jax.ShapeDtypeStruct((_NP, 128), jnp.float32),
        ],
    )(x_pad, W1, ms, md)

    acc1 = _sc_layer(a1, btab1, src, dst, _edge_body1)

    a2, btab2 = pl.pallas_call(
        _tc_combine1,
        grid=grid,
        in_specs=[
            pl.BlockSpec((2, blk, 128), lambda i: (0, i, 0)),
            pl.BlockSpec((1, 64), lambda i: (0, 0)),
            pl.BlockSpec((64, _NCLASS), lambda i: (0, 0)),
            pl.BlockSpec((_NCLASS, 16), lambda i: (0, 0)),
            pl.BlockSpec((_NCLASS, 16), lambda i: (0, 0)),
        ],
        out_specs=[
            pl.BlockSpec((blk, 128), lambda i: (i, 0)),
            pl.BlockSpec((blk, 128), lambda i: (i, 0)),
        ],
        out_shape=[
            jax.ShapeDtypeStruct((_NP, 128), jnp.float32),
            jax.ShapeDtypeStruct((_NP, 128), jnp.float32),
        ],
    )(acc1, b1.reshape(1, 64), W2, s2, d2)

    acc2 = _sc_layer(a2, btab2, src, dst, _edge_body2)

    out = pl.pallas_call(
        _tc_combine2,
        grid=grid,
        in_specs=[
            pl.BlockSpec((2, blk, 128), lambda i: (0, i, 0)),
            pl.BlockSpec((1, _NCLASS), lambda i: (0, 0)),
        ],
        out_specs=pl.BlockSpec((blk, _NCLASS), lambda i: (i, 0)),
        out_shape=jax.ShapeDtypeStruct((_NP, _NCLASS), jnp.float32),
    )(acc2, b2.reshape(1, _NCLASS))

    return out[:n]


# speedup vs baseline: 39.5102x; 1.0244x over previous
import functools

import jax
import jax.numpy as jnp
from jax.experimental import pallas as pl
from jax.experimental.pallas import tpu as pltpu
from jax.experimental.pallas import tpu_sc as plsc

_N = 10000
_E = 320000
_NFEAT = 128
_NCLASS = 40
_H1, _C1 = 8, 8

_NC, _NS = 2, 16
_NW = _NC * _NS
_CHUNK = 128
_ROWS_PER_TILE = 640
_NP = _NW * _ROWS_PER_TILE // 2
_ETOT = _E + _N
_CHUNKS_PER_TILE = 2 * (-(-_ETOT // (2 * _NW * _CHUNK)))
_NCHUNK = _NW * _CHUNKS_PER_TILE
_EP = _NCHUNK * _CHUNK


def _tc_tables1(x_ref, w1_ref, ms_ref, md_ref, a_ref, b_ref):
    h = jnp.dot(x_ref[...], w1_ref[...], preferred_element_type=jnp.float32)
    asb = jnp.dot(h, ms_ref[...], preferred_element_type=jnp.float32)
    adb = jnp.dot(h, md_ref[...], preferred_element_type=jnp.float32)
    a_ref[...] = jnp.concatenate([h, asb], axis=1)
    b_ref[...] = jnp.concatenate(
        [adb, jnp.zeros((adb.shape[0], 64), jnp.float32)], axis=1)


def _tc_combine1(acc_ref, b1_ref, w2_ref, s2_ref, d2_ref, a_ref, b_ref):
    a = acc_ref[0] + acc_ref[1]
    num = a[:, :64]
    den = a[:, 64:]
    x1 = num / (den + 1e-16) + b1_ref[...]
    x1 = jnp.where(x1 > 0, x1, jnp.exp(jnp.minimum(x1, 0.0)) - 1.0)
    h2 = jnp.dot(x1, w2_ref[...], preferred_element_type=jnp.float32)
    as2 = jnp.dot(h2, s2_ref[...], preferred_element_type=jnp.float32)
    ad2 = jnp.dot(h2, d2_ref[...], preferred_element_type=jnp.float32)
    zeros8 = jnp.zeros((h2.shape[0], 8), jnp.float32)
    zeros64 = jnp.zeros((h2.shape[0], 64), jnp.float32)
    a_ref[...] = jnp.concatenate([h2, zeros8, as2, zeros64], axis=1)
    b_ref[...] = jnp.concatenate(
        [ad2, jnp.zeros((ad2.shape[0], 112), jnp.float32)], axis=1)


def _tc_combine2(acc_ref, b2_ref, o_ref):
    a = acc_ref[0] + acc_ref[1]
    num = a[:, :_NCLASS]
    den = a[:, 48:49]
    o_ref[...] = num / (den + 1e-16) + b2_ref[...]


def _edge_body1(ga, gb, e):
    for p in range(4):
        va = ga[e, pl.ds(64 + 16 * p, 16)]
        vd = gb[e, pl.ds(16 * p, 16)]
        s = va + vd
        w = jnp.exp(jnp.where(s > 0, s, 0.2 * s))
        ga[e, pl.ds(16 * p, 16)] = ga[e, pl.ds(16 * p, 16)] * w
        ga[e, pl.ds(64 + 16 * p, 16)] = w


def _edge_body2(ga, gb, e):
    s = ga[e, pl.ds(48, 16)] + gb[e, pl.ds(0, 16)]
    w = jnp.exp(jnp.where(s > 0, s, 0.2 * s))
    for p in range(3):
        ga[e, pl.ds(16 * p, 16)] = ga[e, pl.ds(16 * p, 16)] * w
    ga[e, pl.ds(48, 16)] = w


def _sc_edge_pass(a_hbm, b_hbm, src_hbm, dst_hbm, zero_hbm, out_hbm,
                  sidx0, sidx1, didx0, didx1, ga, gb, acc,
                  gsa, gsb, is0, is1, *, edge_body):
    cid = jax.lax.axis_index("c")
    sid = jax.lax.axis_index("s")
    wid = sid * _NC + cid

    pltpu.sync_copy(zero_hbm, acc.at[pl.ds(sid * _ROWS_PER_TILE, _ROWS_PER_TILE)])
    c0 = wid * _CHUNKS_PER_TILE
    pltpu.sync_copy(src_hbm.at[c0], sidx0)
    pltpu.sync_copy(dst_hbm.at[c0], didx0)
    plsc.subcore_barrier()

    idxbufs = ((sidx0, didx0, is0), (sidx1, didx1, is1))

    @pl.loop(0, _CHUNKS_PER_TILE // 2)
    def _(i):
        for b in range(2):
            sidx, didx, isem = idxbufs[b]
            nsidx, ndidx, nisem = idxbufs[1 - b]
            j = 2 * i + b
            @pl.when(j >= 1)
            def _():
                c = wid * _CHUNKS_PER_TILE + j
                pltpu.make_async_copy(src_hbm.at[c], sidx, isem).wait()
                pltpu.make_async_copy(dst_hbm.at[c], didx, isem).wait()

            ca = pltpu.async_copy(a_hbm.at[sidx], ga, gsa)
            cb = pltpu.async_copy(b_hbm.at[didx], gb, gsb)

            @pl.when(j + 1 < _CHUNKS_PER_TILE)
            def _():
                c1 = wid * _CHUNKS_PER_TILE + j + 1
                pltpu.async_copy(src_hbm.at[c1], nsidx, nisem)
                pltpu.async_copy(dst_hbm.at[c1], ndidx, nisem)

            ca.wait()
            cb.wait()

            @pl.loop(0, _CHUNK)
            def _(e):
                edge_body(ga, gb, e)

            pltpu.sync_copy(ga, acc.at[didx], add=True)

    plsc.subcore_barrier()
    pltpu.sync_copy(
        acc.at[pl.ds(sid * _ROWS_PER_TILE, _ROWS_PER_TILE)],
        out_hbm.at[pl.ds((cid * _NP + sid * _ROWS_PER_TILE), _ROWS_PER_TILE)],
    )


def _sc_layer(a_tab, b_tab, src2d, dst2d, edge_body):
    zero = jnp.zeros((_ROWS_PER_TILE, 128), jnp.float32)
    mesh = plsc.VectorSubcoreMesh(core_axis_name="c", subcore_axis_name="s")
    kern = pl.kernel(
        functools.partial(_sc_edge_pass, edge_body=edge_body),
        out_type=jax.ShapeDtypeStruct((2 * _NP, 128), jnp.float32),
        mesh=mesh,
        scratch_types=[
            pltpu.VMEM((_CHUNK,), jnp.int32),
            pltpu.VMEM((_CHUNK,), jnp.int32),
            pltpu.VMEM((_CHUNK,), jnp.int32),
            pltpu.VMEM((_CHUNK,), jnp.int32),
            pltpu.VMEM((_CHUNK, 128), jnp.float32),
            pltpu.VMEM((_CHUNK, 128), jnp.float32),
            pltpu.VMEM_SHARED((_NP, 128), jnp.float32),
            pltpu.SemaphoreType.DMA,
            pltpu.SemaphoreType.DMA,
            pltpu.SemaphoreType.DMA,
            pltpu.SemaphoreType.DMA,
        ],
    )
    out = kern(a_tab, b_tab, src2d, dst2d, zero)
    return out.reshape(2, _NP, 128)


def kernel(x, edge_index, W1, a_src1, a_dst1, b1, W2, a_src2, a_dst2, b2):
    n = x.shape[0]
    loops = jnp.arange(n, dtype=jnp.int32)
    padi = jnp.full((_EP - _ETOT,), n, dtype=jnp.int32)
    src = jnp.concatenate([edge_index[0], loops, padi]).reshape(_NCHUNK, _CHUNK)
    dst = jnp.concatenate([edge_index[1], loops, padi]).reshape(_NCHUNK, _CHUNK)

    x_pad = jnp.zeros((_NP, _NFEAT), jnp.float32).at[:n].set(x)

    lane64 = jnp.arange(64)
    headmask = (lane64[:, None] // _C1 == lane64[None, :] // _C1)
    ms = jnp.where(headmask, a_src1.reshape(64)[:, None], 0.0)
    md = jnp.where(headmask, a_dst1.reshape(64)[:, None], 0.0)
    s2 = jnp.broadcast_to(a_src2[0][:, None], (_NCLASS, 16)).astype(jnp.float32)
    d2 = jnp.broadcast_to(a_dst2[0][:, None], (_NCLASS, 16)).astype(jnp.float32)

    blk = 512
    grid = (_NP // blk,)

    a1, btab1 = pl.pallas_call(
        _tc_tables1,
        grid=grid,
        in_specs=[
            pl.BlockSpec((blk, _NFEAT), lambda i: (i, 0)),
            pl.BlockSpec((_NFEAT, 64), lambda i: (0, 0)),
            pl.BlockSpec((64, 64), lambda i: (0, 0)),
            pl.BlockSpec((64, 64), lambda i: (0, 0)),
        ],
        out_specs=[
            pl.BlockSpec((blk, 128), lambda i: (i, 0)),
            pl.BlockSpec((blk, 128), lambda i: (i, 0)),
        ],
        out_shape=[
            jax.ShapeDtypeStruct((_NP, 128), jnp.float32),
            jax.ShapeDtypeStruct((_NP, 128), jnp.float32),
        ],
    )(x_pad, W1, ms, md)

    acc1 = _sc_layer(a1, btab1, src, dst, _edge_body1)

    a2, btab2 = pl.pallas_call(
        _tc_combine1,
        grid=grid,
        in_specs=[
            pl.BlockSpec((2, blk, 128), lambda i: (0, i, 0)),
            pl.BlockSpec((1, 64), lambda i: (0, 0)),
            pl.BlockSpec((64, _NCLASS), lambda i: (0, 0)),
            pl.BlockSpec((_NCLASS, 16), lambda i: (0, 0)),
            pl.BlockSpec((_NCLASS, 16), lambda i: (0, 0)),
        ],
        out_specs=[
            pl.BlockSpec((blk, 128), lambda i: (i, 0)),
            pl.BlockSpec((blk, 128), lambda i: (i, 0)),
        ],
        out_shape=[
            jax.ShapeDtypeStruct((_NP, 128), jnp.float32),
            jax.ShapeDtypeStruct((_NP, 128), jnp.float32),
        ],
    )(acc1, b1.reshape(1, 64), W2, s2, d2)

    acc2 = _sc_layer(a2, btab2, src, dst, _edge_body2)

    out = pl.pallas_call(
        _tc_combine2,
        grid=grid,
        in_specs=[
            pl.BlockSpec((2, blk, 128), lambda i: (0, i, 0)),
            pl.BlockSpec((1, _NCLASS), lambda i: (0, 0)),
        ],
        out_specs=pl.BlockSpec((blk, _NCLASS), lambda i: (i, 0)),
        out_shape=jax.ShapeDtypeStruct((_NP, _NCLASS), jnp.float32),
    )(acc2, b2.reshape(1, _NCLASS))

    return out[:n]


# speedup vs baseline: 60.7974x; 1.5388x over previous
import functools

import jax
import jax.numpy as jnp
from jax.experimental import pallas as pl
from jax.experimental.pallas import tpu as pltpu
from jax.experimental.pallas import tpu_sc as plsc

_N = 10000
_E = 320000
_NFEAT = 128
_NCLASS = 40
_H1, _C1 = 8, 8

_NC, _NS = 2, 16
_NW = _NC * _NS
_CHUNK = 80
_ROWS_PER_TILE = 640
_NP = _NW * _ROWS_PER_TILE // 2
_ETOT = _E + _N
_CHUNKS_PER_TILE = 2 * (-(-_ETOT // (2 * _NW * _CHUNK)))
_NCHUNK = _NW * _CHUNKS_PER_TILE
_EP = _NCHUNK * _CHUNK


def _tc_tables1(x_ref, w1_ref, ms_ref, md_ref, a_ref, b_ref):
    h = jnp.dot(x_ref[...], w1_ref[...], preferred_element_type=jnp.float32)
    asb = jnp.dot(h, ms_ref[...], preferred_element_type=jnp.float32)
    adb = jnp.dot(h, md_ref[...], preferred_element_type=jnp.float32)
    a_ref[...] = jnp.concatenate([h, asb], axis=1)
    b_ref[...] = jnp.concatenate(
        [adb, jnp.zeros((adb.shape[0], 64), jnp.float32)], axis=1)


def _tc_combine1(acc_ref, b1_ref, w2_ref, s2_ref, d2_ref, a_ref, b_ref):
    a = acc_ref[0] + acc_ref[1]
    num = a[:, :64]
    den = a[:, 64:]
    x1 = num / (den + 1e-16) + b1_ref[...]
    x1 = jnp.where(x1 > 0, x1, jnp.exp(jnp.minimum(x1, 0.0)) - 1.0)
    h2 = jnp.dot(x1, w2_ref[...], preferred_element_type=jnp.float32)
    as2 = jnp.dot(h2, s2_ref[...], preferred_element_type=jnp.float32)
    ad2 = jnp.dot(h2, d2_ref[...], preferred_element_type=jnp.float32)
    zeros8 = jnp.zeros((h2.shape[0], 8), jnp.float32)
    zeros64 = jnp.zeros((h2.shape[0], 64), jnp.float32)
    a_ref[...] = jnp.concatenate([h2, zeros8, as2, zeros64], axis=1)
    b_ref[...] = jnp.concatenate(
        [ad2, jnp.zeros((ad2.shape[0], 112), jnp.float32)], axis=1)


def _tc_combine2(acc_ref, b2_ref, o_ref):
    a = acc_ref[0] + acc_ref[1]
    num = a[:, :_NCLASS]
    den = a[:, 48:49]
    o_ref[...] = num / (den + 1e-16) + b2_ref[...]


def _edge_body1(ga, gb, e):
    for p in range(4):
        va = ga[e, pl.ds(64 + 16 * p, 16)]
        vd = gb[e, pl.ds(16 * p, 16)]
        s = va + vd
        w = jnp.exp(jnp.maximum(s, 0.2 * s))
        ga[e, pl.ds(16 * p, 16)] = ga[e, pl.ds(16 * p, 16)] * w
        ga[e, pl.ds(64 + 16 * p, 16)] = w


def _edge_body2(ga, gb, e):
    s = ga[e, pl.ds(48, 16)] + gb[e, pl.ds(0, 16)]
    w = jnp.exp(jnp.maximum(s, 0.2 * s))
    for p in range(3):
        ga[e, pl.ds(16 * p, 16)] = ga[e, pl.ds(16 * p, 16)] * w
    ga[e, pl.ds(48, 16)] = w


def _sc_edge_pass(a_hbm, b_hbm, src_hbm, dst_hbm, zero_hbm, out_hbm,
                  sidx0, sidx1, didx0, didx1, ga0, ga1, gb0, gb1, acc,
                  gsa0, gsb0, gsa1, gsb1, is0, is1, *, edge_body):
    cid = jax.lax.axis_index("c")
    sid = jax.lax.axis_index("s")
    wid = sid * _NC + cid

    pltpu.sync_copy(zero_hbm, acc.at[pl.ds(sid * _ROWS_PER_TILE, _ROWS_PER_TILE)])
    bufs = ((sidx0, didx0, ga0, gb0, gsa0, gsb0, is0),
            (sidx1, didx1, ga1, gb1, gsa1, gsb1, is1))
    c0 = wid * _CHUNKS_PER_TILE
    for b in range(2):
        sidx, didx, ga, gb, gsa, gsb, _ = bufs[b]
        pltpu.sync_copy(src_hbm.at[c0 + b], sidx)
        pltpu.sync_copy(dst_hbm.at[c0 + b], didx)
        pltpu.async_copy(a_hbm.at[sidx], ga, gsa)
        pltpu.async_copy(b_hbm.at[didx], gb, gsb)
    plsc.subcore_barrier()

    @pl.loop(0, _CHUNKS_PER_TILE // 2)
    def _(i):
        for b in range(2):
            sidx, didx, ga, gb, gsa, gsb, isem = bufs[b]
            j = 2 * i + b
            pltpu.make_async_copy(a_hbm.at[sidx], ga, gsa).wait()
            pltpu.make_async_copy(b_hbm.at[didx], gb, gsb).wait()

            @pl.loop(0, _CHUNK)
            def _(e):
                edge_body(ga, gb, e)

            pltpu.sync_copy(ga, acc.at[didx], add=True)

            @pl.when(j + 2 < _CHUNKS_PER_TILE)
            def _():
                c = wid * _CHUNKS_PER_TILE + j + 2
                pltpu.async_copy(src_hbm.at[c], sidx, isem)
                pltpu.async_copy(dst_hbm.at[c], didx, isem)
                pltpu.make_async_copy(src_hbm.at[c], sidx, isem).wait()
                pltpu.make_async_copy(dst_hbm.at[c], didx, isem).wait()
                pltpu.async_copy(a_hbm.at[sidx], ga, gsa)
                pltpu.async_copy(b_hbm.at[didx], gb, gsb)

    plsc.subcore_barrier()
    pltpu.sync_copy(
        acc.at[pl.ds(sid * _ROWS_PER_TILE, _ROWS_PER_TILE)],
        out_hbm.at[pl.ds((cid * _NP + sid * _ROWS_PER_TILE), _ROWS_PER_TILE)],
    )


def _sc_layer(a_tab, b_tab, src2d, dst2d, edge_body):
    zero = jnp.zeros((_ROWS_PER_TILE, 128), jnp.float32)
    mesh = plsc.VectorSubcoreMesh(core_axis_name="c", subcore_axis_name="s")
    kern = pl.kernel(
        functools.partial(_sc_edge_pass, edge_body=edge_body),
        out_type=jax.ShapeDtypeStruct((2 * _NP, 128), jnp.float32),
        mesh=mesh,
        scratch_types=[
            pltpu.VMEM((_CHUNK,), jnp.int32),
            pltpu.VMEM((_CHUNK,), jnp.int32),
            pltpu.VMEM((_CHUNK,), jnp.int32),
            pltpu.VMEM((_CHUNK,), jnp.int32),
            pltpu.VMEM((_CHUNK, 128), jnp.float32),
            pltpu.VMEM((_CHUNK, 128), jnp.float32),
            pltpu.VMEM((_CHUNK, 128), jnp.float32),
            pltpu.VMEM((_CHUNK, 128), jnp.float32),
            pltpu.VMEM_SHARED((_NP, 128), jnp.float32),
            pltpu.SemaphoreType.DMA,
            pltpu.SemaphoreType.DMA,
            pltpu.SemaphoreType.DMA,
            pltpu.SemaphoreType.DMA,
            pltpu.SemaphoreType.DMA,
            pltpu.SemaphoreType.DMA,
        ],
    )
    out = kern(a_tab, b_tab, src2d, dst2d, zero)
    return out.reshape(2, _NP, 128)


def kernel(x, edge_index, W1, a_src1, a_dst1, b1, W2, a_src2, a_dst2, b2):
    n = x.shape[0]
    loops = jnp.arange(n, dtype=jnp.int32)
    padi = jnp.full((_EP - _ETOT,), n, dtype=jnp.int32)
    src = jnp.concatenate([edge_index[0], loops, padi]).reshape(_NCHUNK, _CHUNK)
    dst = jnp.concatenate([edge_index[1], loops, padi]).reshape(_NCHUNK, _CHUNK)

    x_pad = jnp.zeros((_NP, _NFEAT), jnp.float32).at[:n].set(x)

    lane64 = jnp.arange(64)
    headmask = (lane64[:, None] // _C1 == lane64[None, :] // _C1)
    ms = jnp.where(headmask, a_src1.reshape(64)[:, None], 0.0)
    md = jnp.where(headmask, a_dst1.reshape(64)[:, None], 0.0)
    s2 = jnp.broadcast_to(a_src2[0][:, None], (_NCLASS, 16)).astype(jnp.float32)
    d2 = jnp.broadcast_to(a_dst2[0][:, None], (_NCLASS, 16)).astype(jnp.float32)

    blk = 512
    grid = (_NP // blk,)

    a1, btab1 = pl.pallas_call(
        _tc_tables1,
        grid=grid,
        in_specs=[
            pl.BlockSpec((blk, _NFEAT), lambda i: (i, 0)),
            pl.BlockSpec((_NFEAT, 64), lambda i: (0, 0)),
            pl.BlockSpec((64, 64), lambda i: (0, 0)),
            pl.BlockSpec((64, 64), lambda i: (0, 0)),
        ],
        out_specs=[
            pl.BlockSpec((blk, 128), lambda i: (i, 0)),
            pl.BlockSpec((blk, 128), lambda i: (i, 0)),
        ],
        out_shape=[
            jax.ShapeDtypeStruct((_NP, 128), jnp.float32),
            jax.ShapeDtypeStruct((_NP, 128), jnp.float32),
        ],
    )(x_pad, W1, ms, md)

    acc1 = _sc_layer(a1, btab1, src, dst, _edge_body1)

    a2, btab2 = pl.pallas_call(
        _tc_combine1,
        grid=grid,
        in_specs=[
            pl.BlockSpec((2, blk, 128), lambda i: (0, i, 0)),
            pl.BlockSpec((1, 64), lambda i: (0, 0)),
            pl.BlockSpec((64, _NCLASS), lambda i: (0, 0)),
            pl.BlockSpec((_NCLASS, 16), lambda i: (0, 0)),
            pl.BlockSpec((_NCLASS, 16), lambda i: (0, 0)),
        ],
        out_specs=[
            pl.BlockSpec((blk, 128), lambda i: (i, 0)),
            pl.BlockSpec((blk, 128), lambda i: (i, 0)),
        ],
        out_shape=[
            jax.ShapeDtypeStruct((_NP, 128), jnp.float32),
            jax.ShapeDtypeStruct((_NP, 128), jnp.float32),
        ],
    )(acc1, b1.reshape(1, 64), W2, s2, d2)

    acc2 = _sc_layer(a2, btab2, src, dst, _edge_body2)

    out = pl.pallas_call(
        _tc_combine2,
        grid=grid,
        in_specs=[
            pl.BlockSpec((2, blk, 128), lambda i: (0, i, 0)),
            pl.BlockSpec((1, _NCLASS), lambda i: (0, 0)),
        ],
        out_specs=pl.BlockSpec((blk, _NCLASS), lambda i: (i, 0)),
        out_shape=jax.ShapeDtypeStruct((_NP, _NCLASS), jnp.float32),
    )(acc2, b2.reshape(1, _NCLASS))

    return out[:n]
